# Initial kernel scaffold; baseline (speedup 1.0000x reference)
import jax
import jax.numpy as jnp
from jax.experimental import pallas as pl


def kernel(sim_mat, x):
    raise NotImplementedError("write your pallas kernel here")


# speedup vs baseline: 15.4609x; 15.4609x over previous
import functools

import jax
import jax.numpy as jnp
from jax import lax
from jax.experimental import pallas as pl
from jax.experimental.pallas import tpu as pltpu
from jax.experimental.pallas import tpu_sc as plsc

B = 4
C = 192
HW = 128 * 128
NC = 2
NS = 16
NW = NC * NS
L = 16
P = 256
PIX_PER_W = (B * HW) // NW
CHUNKS = PIX_PER_W // P
UNROLL = 8


def _sc_kernel(sim_hbm, out_hbm, buf, outb):
    cid = lax.axis_index("c")
    sid = lax.axis_index("s")
    wid = sid * NC + cid
    img = wid // (NW // B)
    col0 = (wid % (NW // B)) * PIX_PER_W

    def chunk_body(k, _):
        col = col0 + k * P
        pltpu.sync_copy(sim_hbm.at[img, :, pl.ds(col, P)], buf)

        def group_body(j, _):
            base = j * L
            neg = jnp.full((L,), -jnp.inf, jnp.float32)

            def step(i, carry):
                m1, m2, s = carry
                for u in range(UNROLL):
                    v = buf[i * UNROLL + u, pl.ds(base, L)]
                    s = s + jnp.exp(v)
                    nm1 = jnp.maximum(m1, v)
                    m2 = jnp.maximum(m2, jnp.minimum(m1, v))
                    m1 = nm1
                return m1, m2, s

            m1, m2, s = lax.fori_loop(
                0, C // UNROLL, step,
                (neg, neg, jnp.zeros((L,), jnp.float32)))
            ratio = jnp.exp(m1) / (jnp.exp(m2) + 1e-8 * s)
            outb[pl.ds(base, L)] = jnp.exp(1.0 - ratio)
            return 0

        lax.fori_loop(0, P // L, group_body, 0)
        pltpu.sync_copy(outb, out_hbm.at[img, pl.ds(col, P)])
        return 0

    lax.fori_loop(0, CHUNKS, chunk_body, 0)


@jax.jit
def _run(sim3):
    mesh = plsc.VectorSubcoreMesh(core_axis_name="c", subcore_axis_name="s")
    fn = pl.kernel(
        _sc_kernel,
        out_type=jax.ShapeDtypeStruct((B, HW), jnp.float32),
        mesh=mesh,
        scratch_types=[
            pltpu.VMEM((C, P), jnp.float32),
            pltpu.VMEM((P,), jnp.float32),
        ],
    )
    return fn(sim3)


def kernel(sim_mat, x):
    del x
    sim3 = sim_mat.reshape(B, C, HW)
    out = _run(sim3)
    return (out.reshape(B, 128, 128),)


# speedup vs baseline: 20.5892x; 1.3317x over previous
import jax
import jax.numpy as jnp
from jax import lax
from jax.experimental import pallas as pl
from jax.experimental.pallas import tpu as pltpu
from jax.experimental.pallas import tpu_sc as plsc

B = 4
C = 192
HW = 128 * 128
NC = 2
NS = 16
NW = NC * NS
L = 16
P = 256
PIX_PER_W = (B * HW) // NW
CHUNKS = PIX_PER_W // P
UNROLL = 8


def _sc_kernel(sim_hbm, out_hbm, buf0, buf1, outb, sem0, sem1):
    cid = lax.axis_index("c")
    sid = lax.axis_index("s")
    wid = sid * NC + cid
    img = wid // (NW // B)
    col0 = (wid % (NW // B)) * PIX_PER_W

    def src(k):
        return sim_hbm.at[img, :, pl.ds(col0 + k * P, P)]

    def compute(buf, k):
        def group_body(j, _):
            base = j * (2 * L)
            neg = jnp.full((L,), -jnp.inf, jnp.float32)

            def step(i, carry):
                a1, a2, b1, b2 = carry
                for u in range(UNROLL):
                    c = i * UNROLL + u
                    va = buf[c, pl.ds(base, L)]
                    vb = buf[c, pl.ds(base + L, L)]
                    na1 = jnp.maximum(a1, va)
                    a2 = jnp.maximum(a2, jnp.minimum(a1, va))
                    a1 = na1
                    nb1 = jnp.maximum(b1, vb)
                    b2 = jnp.maximum(b2, jnp.minimum(b1, vb))
                    b1 = nb1
                return a1, a2, b1, b2

            a1, a2, b1, b2 = lax.fori_loop(
                0, C // UNROLL, step, (neg, neg, neg, neg))
            o = k * P + base
            outb[pl.ds(o, L)] = jnp.exp(1.0 - jnp.exp(a1 - a2))
            outb[pl.ds(o + L, L)] = jnp.exp(1.0 - jnp.exp(b1 - b2))
            return 0

        lax.fori_loop(0, P // (2 * L), group_body, 0)

    pltpu.async_copy(src(0), buf0, sem0)
    pltpu.async_copy(src(1), buf1, sem1)

    def pair(g, _):
        k0 = 2 * g
        pltpu.make_async_copy(src(k0), buf0, sem0).wait()
        compute(buf0, k0)

        @pl.when(k0 + 2 < CHUNKS)
        def _():
            pltpu.async_copy(src(k0 + 2), buf0, sem0)

        k1 = k0 + 1
        pltpu.make_async_copy(src(k1), buf1, sem1).wait()
        compute(buf1, k1)

        @pl.when(k1 + 2 < CHUNKS)
        def _():
            pltpu.async_copy(src(k1 + 2), buf1, sem1)

        return 0

    lax.fori_loop(0, CHUNKS // 2, pair, 0)
    pltpu.sync_copy(outb, out_hbm.at[img, pl.ds(col0, PIX_PER_W)])


@jax.jit
def _run(sim3):
    mesh = plsc.VectorSubcoreMesh(core_axis_name="c", subcore_axis_name="s")
    fn = pl.kernel(
        _sc_kernel,
        out_type=jax.ShapeDtypeStruct((B, HW), jnp.float32),
        mesh=mesh,
        scratch_types=[
            pltpu.VMEM((C, P), jnp.float32),
            pltpu.VMEM((C, P), jnp.float32),
            pltpu.VMEM((PIX_PER_W,), jnp.float32),
            pltpu.SemaphoreType.DMA,
            pltpu.SemaphoreType.DMA,
        ],
    )
    return fn(sim3)


def kernel(sim_mat, x):
    del x
    sim3 = sim_mat.reshape(B, C, HW)
    out = _run(sim3)
    return (out.reshape(B, 128, 128),)


# speedup vs baseline: 44.2946x; 2.1513x over previous
import jax
import jax.numpy as jnp
from jax import lax
from jax.experimental import pallas as pl
from jax.experimental.pallas import tpu as pltpu
from jax.experimental.pallas import tpu_sc as plsc

B = 4
H = 128
W = 128
C = 192
NC = 2
NS = 16
NW = NC * NS
L = 16
ROWS_PER_W = (B * H) // NW
RPC = 2
CHUNKS = ROWS_PER_W // RPC
UNROLL = 8


def _sc_kernel(sim_hbm, out_hbm, buf0, buf1, outb, sem0, sem1):
    cid = lax.axis_index("c")
    sid = lax.axis_index("s")
    wid = sid * NC + cid
    img = wid // (NW // B)
    row0 = (wid % (NW // B)) * ROWS_PER_W

    def src(k):
        return sim_hbm.at[img, :, pl.ds(row0 + k * RPC, RPC), :]

    def compute(buf, k):
        for r in range(RPC):
            def group_body(j, _, r=r):
                base = j * (2 * L)
                neg = jnp.full((L,), -jnp.inf, jnp.float32)

                def step(i, carry):
                    a1, a2, b1, b2 = carry
                    for u in range(UNROLL):
                        c = i * UNROLL + u
                        va = buf[c, r, pl.ds(base, L)]
                        vb = buf[c, r, pl.ds(base + L, L)]
                        na1 = jnp.maximum(a1, va)
                        a2 = jnp.maximum(a2, jnp.minimum(a1, va))
                        a1 = na1
                        nb1 = jnp.maximum(b1, vb)
                        b2 = jnp.maximum(b2, jnp.minimum(b1, vb))
                        b1 = nb1
                    return a1, a2, b1, b2

                a1, a2, b1, b2 = lax.fori_loop(
                    0, C // UNROLL, step, (neg, neg, neg, neg))
                row = k * RPC + r
                outb[row, pl.ds(base, L)] = jnp.exp(1.0 - jnp.exp(a1 - a2))
                outb[row, pl.ds(base + L, L)] = jnp.exp(1.0 - jnp.exp(b1 - b2))
                return 0

            lax.fori_loop(0, W // (2 * L), group_body, 0)

    pltpu.async_copy(src(0), buf0, sem0)
    pltpu.async_copy(src(1), buf1, sem1)

    def pair(g, _):
        k0 = 2 * g
        pltpu.make_async_copy(src(k0), buf0, sem0).wait()
        compute(buf0, k0)

        @pl.when(k0 + 2 < CHUNKS)
        def _():
            pltpu.async_copy(src(k0 + 2), buf0, sem0)

        k1 = k0 + 1
        pltpu.make_async_copy(src(k1), buf1, sem1).wait()
        compute(buf1, k1)

        @pl.when(k1 + 2 < CHUNKS)
        def _():
            pltpu.async_copy(src(k1 + 2), buf1, sem1)

        return 0

    lax.fori_loop(0, CHUNKS // 2, pair, 0)
    pltpu.sync_copy(outb, out_hbm.at[img, pl.ds(row0, ROWS_PER_W), :])


@jax.jit
def _run(sim_mat):
    mesh = plsc.VectorSubcoreMesh(core_axis_name="c", subcore_axis_name="s")
    fn = pl.kernel(
        _sc_kernel,
        out_type=jax.ShapeDtypeStruct((B, H, W), jnp.float32),
        mesh=mesh,
        scratch_types=[
            pltpu.VMEM((C, RPC, W), jnp.float32),
            pltpu.VMEM((C, RPC, W), jnp.float32),
            pltpu.VMEM((ROWS_PER_W, W), jnp.float32),
            pltpu.SemaphoreType.DMA,
            pltpu.SemaphoreType.DMA,
        ],
    )
    return fn(sim_mat)


def kernel(sim_mat, x):
    del x
    return (_run(sim_mat),)
